# baseline (speedup 1.0000x reference)
import functools

import jax
import jax.numpy as jnp
from jax import lax
from jax.experimental import pallas as pl
from jax.experimental.pallas import tpu as pltpu
from jax.experimental.pallas import tpu_sc as plsc

D = 16
OUT = 2
HIST = 20
L = 16
NC, NS = 2, 16
NW = NC * NS
B = 16384
BPW = B // NW
CHUNK = 128
NCHUNK = BPW // CHUNK
GROUPS = BPW // L


def _tok_router(ids_hbm, table_hbm, wb_hbm, out_hbm,
                ids_v, idx_v, rows_v, wb_v, out_v, sem):
    wid = lax.axis_index("s") * NC + lax.axis_index("c")
    base = wid * BPW

    pltpu.sync_copy(ids_hbm.at[pl.ds(base * HIST, BPW * HIST)], ids_v)
    pltpu.sync_copy(wb_hbm, wb_v)

    iota = lax.iota(jnp.int32, L)

    for g in range(GROUPS):
        ridx = iota + (g * L)
        col0 = plsc.load_gather(ids_v, [ridx * HIST])
        plsc.store_scatter(idx_v, [ridx], col0)

    copies = [
        pltpu.async_copy(table_hbm.at[idx_v.at[pl.ds(c * CHUNK, CHUNK)]],
                         rows_v.at[pl.ds(c * CHUNK, CHUNK)], sem)
        for c in range(NCHUNK)
    ]
    for cp in copies:
        cp.wait()

    w_splat = [[wb_v[pl.ds((k * OUT + j) * L, L)] for j in range(OUT)]
               for k in range(D)]
    b_splat = [wb_v[pl.ds((D * OUT + j) * L, L)] for j in range(OUT)]

    def group_body(g, carry):
        ridx = iota + g * L
        acc0 = b_splat[0]
        acc1 = b_splat[1]
        for k in range(D):
            v = plsc.load_gather(rows_v, [ridx, jnp.full((L,), k, jnp.int32)])
            acc0 = acc0 + v * w_splat[k][0]
            acc1 = acc1 + v * w_splat[k][1]
        obase = ridx * OUT
        plsc.store_scatter(out_v, [obase], acc0)
        plsc.store_scatter(out_v, [obase + 1], acc1)
        return carry

    lax.fori_loop(0, GROUPS, group_body, 0)

    pltpu.sync_copy(out_v, out_hbm.at[pl.ds(base * OUT, BPW * OUT)])


def kernel(ids, table, W, b):
    wb = jnp.broadcast_to(
        jnp.concatenate([W.reshape(-1).astype(jnp.float32),
                         b.astype(jnp.float32)])[:, None],
        (D * OUT + OUT, 16)).reshape(-1)
    mesh = plsc.VectorSubcoreMesh(core_axis_name="c", subcore_axis_name="s")
    fn = functools.partial(
        pl.kernel,
        mesh=mesh,
        compiler_params=pltpu.CompilerParams(use_tc_tiling_on_sc=False,
                                             needs_layout_passes=False),
        out_type=jax.ShapeDtypeStruct((B * OUT,), jnp.float32),
        scratch_types=[
            pltpu.VMEM((BPW * HIST,), jnp.int32),
            pltpu.VMEM((BPW,), jnp.int32),
            pltpu.VMEM((BPW, D), jnp.float32),
            pltpu.VMEM(((D * OUT + OUT) * 16,), jnp.float32),
            pltpu.VMEM((BPW * OUT,), jnp.float32),
            pltpu.SemaphoreType.DMA,
        ],
    )(_tok_router)
    out_flat = fn(ids.reshape(-1).astype(jnp.int32), table, wb)
    return out_flat.reshape(B, OUT)
